# baseline (speedup 1.0000x reference)
import functools

import jax
import jax.numpy as jnp
from jax import lax
from jax.experimental import pallas as pl
from jax.experimental.pallas import tpu as pltpu
from jax.experimental.pallas import tpu_sc as plsc

B = 16384
F = 26
V = 100001
D = 32

NC = 2
NS = 16
NW = NC * NS
VEC_PW = F * D // NW

QCH = 4096
NQ = B // QCH

VCH = 12544
NVCH = 7
VLAST = VCH


def _emb_body(tab_hbm, idx_hbm, out_hbm, vvec, idxv, outv, vsem, osem):
  wid = lax.axis_index("s") * NC + lax.axis_index("c")

  def vec_body(k, f_prev):
    f = k
    d = wid

    dt = lax.rem(wid, 4) * 8
    off = lax.rem(k, 7) * VCH
    src = tab_hbm.at[f, pl.ds(dt, 8), pl.ds(off, VCH)]
    pltpu.async_copy(src, vvec, vsem)
    pltpu.sync_copy(idx_hbm.at[f], idxv)
    pltpu.make_async_copy(src, vvec, vsem).wait()

    for q in range(NQ):
      buf = q % 2

      @pl.when(k * NQ + q >= 2)
      def _(buf=buf):
        pltpu.make_async_copy(
            outv.at[buf], out_hbm.at[f, d, pl.ds(0, QCH)], osem).wait()

      zero16 = lax.iota(jnp.int32, 16) * 0

      def qstep(i, _, q=q, buf=buf):
        base = i * 256
        for j in range(16):
          idx16 = lax.rem(idxv[pl.ds(q * QCH + base + j * 16, 16)], VCH)
          outv[buf, pl.ds(base + j * 16, 16)] = plsc.load_gather(
              vvec, [zero16, idx16])
        return 0

      lax.fori_loop(0, QCH // 256, qstep, 0)
      pltpu.async_copy(
          outv.at[buf], out_hbm.at[f, d, pl.ds(q * QCH, QCH)], osem)
    return f

  lax.fori_loop(0, VEC_PW, vec_body, jnp.int32(-1))
  for _ in range(2):
    pltpu.make_async_copy(
        outv.at[0], out_hbm.at[0, 0, pl.ds(0, QCH)], osem).wait()


@jax.jit
def kernel(indices, tables):
  tabT = jnp.transpose(tables, (0, 2, 1))
  idxT = jnp.transpose(indices, (1, 0))
  mesh = plsc.VectorSubcoreMesh(
      core_axis_name="c", subcore_axis_name="s", num_cores=NC, num_subcores=NS)
  run = functools.partial(
      pl.kernel,
      out_type=jax.ShapeDtypeStruct((F, D, B), jnp.float32),
      mesh=mesh,
      scratch_types=[
          pltpu.VMEM((8, VCH), jnp.float32),
          pltpu.VMEM((B,), jnp.int32),
          pltpu.VMEM((2, QCH), jnp.float32),
          pltpu.SemaphoreType.DMA,
          pltpu.SemaphoreType.DMA,
      ],
      compiler_params=pltpu.CompilerParams(needs_layout_passes=False),
  )(_emb_body)
  outT = run(tabT, idxT)
  return jnp.transpose(outT, (2, 0, 1))


# speedup vs baseline: 3.7975x; 3.7975x over previous
import functools

import jax
import jax.numpy as jnp
from jax import lax
from jax.experimental import pallas as pl
from jax.experimental.pallas import tpu as pltpu
from jax.experimental.pallas import tpu_sc as plsc

B = 16384
F = 26
V = 100001
D = 32

NC = 2
NS = 16
NW = NC * NS
VEC_PW = F * D // NW

QCH = 4096
NQ = B // QCH


def _emb_body(tab_hbm, idx_hbm, out_hbm, vvec, idxv, outv, vsem, osem):
  wid = lax.axis_index("s") * NC + lax.axis_index("c")

  f_prev = None
  nwrites = 0
  for k in range(VEC_PW):
    vid = wid * VEC_PW + k
    f = vid // D
    d = lax.rem(vid, D)

    pltpu.async_copy(tab_hbm.at[f, d], vvec, vsem)
    if f_prev is None:
      pltpu.sync_copy(idx_hbm.at[f], idxv)
    else:
      @pl.when(f != f_prev)
      def _():
        pltpu.sync_copy(idx_hbm.at[f], idxv)
    f_prev = f
    pltpu.make_async_copy(tab_hbm.at[f, d], vvec, vsem).wait()

    for q in range(NQ):
      buf = nwrites % 2
      if nwrites >= 2:
        pltpu.make_async_copy(
            outv.at[buf], out_hbm.at[f, d, pl.ds(0, QCH)], osem).wait()

      def qstep(i, _, q=q, buf=buf):
        base = i * 256
        for j in range(16):
          idx16 = idxv[pl.ds(q * QCH + base + j * 16, 16)]
          outv[buf, pl.ds(base + j * 16, 16)] = plsc.load_gather(
              vvec, [idx16])
        return 0

      lax.fori_loop(0, QCH // 256, qstep, 0)
      pltpu.async_copy(
          outv.at[buf], out_hbm.at[f, d, pl.ds(q * QCH, QCH)], osem)
      nwrites += 1

  for _ in range(2):
    pltpu.make_async_copy(
        outv.at[0], out_hbm.at[0, 0, pl.ds(0, QCH)], osem).wait()


@jax.jit
def kernel(indices, tables):
  tabT = jnp.transpose(tables, (0, 2, 1))
  idxT = jnp.transpose(indices, (1, 0))
  mesh = plsc.VectorSubcoreMesh(
      core_axis_name="c", subcore_axis_name="s", num_cores=NC, num_subcores=NS)
  run = functools.partial(
      pl.kernel,
      out_type=jax.ShapeDtypeStruct((F, D, B), jnp.float32),
      mesh=mesh,
      scratch_types=[
          pltpu.VMEM((V,), jnp.float32),
          pltpu.VMEM((B,), jnp.int32),
          pltpu.VMEM((2, QCH), jnp.float32),
          pltpu.SemaphoreType.DMA,
          pltpu.SemaphoreType.DMA,
      ],
      compiler_params=pltpu.CompilerParams(needs_layout_passes=False),
  )(_emb_body)
  outT = run(tabT, idxT)
  return jnp.transpose(outT, (2, 0, 1))


# speedup vs baseline: 3.8723x; 1.0197x over previous
import functools

import jax
import jax.numpy as jnp
from jax import lax
from jax.experimental import pallas as pl
from jax.experimental.pallas import tpu as pltpu
from jax.experimental.pallas import tpu_sc as plsc

B = 16384
F = 26
V = 100001
D = 32

NC = 2
NS = 16
NW = NC * NS
VEC_PW = F * D // NW

QCH = 4096
NQ = B // QCH


HALF = B // 2


def _emb_body(tab_hbm, idx_hbm, out_hbm, vvec, idxv, outv, vsem):
  wid = lax.axis_index("s") * NC + lax.axis_index("c")

  for k in range(VEC_PW):
    vid = wid * VEC_PW + k
    f = vid // D
    d = lax.rem(vid, D)
    pltpu.async_copy(tab_hbm.at[f, d], vvec, vsem)
    pltpu.sync_copy(idx_hbm.at[f], idxv)
    pltpu.make_async_copy(tab_hbm.at[f, d], vvec, vsem).wait()

    for half in range(2):
      def step(i, _, half=half):
        idx16 = idxv[pl.ds(half * HALF + i * 16, 16)]
        outv[pl.ds(i * 16, 16)] = plsc.load_gather(vvec, [idx16])
        return 0

      lax.fori_loop(0, HALF // 16, step, 0)
      pltpu.sync_copy(outv, out_hbm.at[f, d, pl.ds(half * HALF, HALF)])


@jax.jit
def kernel(indices, tables):
  tabT = jnp.transpose(tables, (0, 2, 1))
  idxT = jnp.transpose(indices, (1, 0))
  mesh = plsc.VectorSubcoreMesh(
      core_axis_name="c", subcore_axis_name="s", num_cores=NC, num_subcores=NS)
  run = functools.partial(
      pl.kernel,
      out_type=jax.ShapeDtypeStruct((F, D, B), jnp.float32),
      mesh=mesh,
      scratch_types=[
          pltpu.VMEM((V,), jnp.float32),
          pltpu.VMEM((B,), jnp.int32),
          pltpu.VMEM((HALF,), jnp.float32),
          pltpu.SemaphoreType.DMA,
      ],
      compiler_params=pltpu.CompilerParams(needs_layout_passes=False),
  )(_emb_body)
  outT = run(tabT, idxT)
  return jnp.transpose(outT, (2, 0, 1))


# speedup vs baseline: 4.9145x; 1.2691x over previous
import functools

import jax
import jax.numpy as jnp
from jax import lax
from jax.experimental import pallas as pl
from jax.experimental.pallas import tpu as pltpu
from jax.experimental.pallas import tpu_sc as plsc

B = 16384
F = 26
V = 100001
D = 32

NC = 2
NS = 16
NW = NC * NS
VEC_PW = F * D // NW

QCH = 4096
NQ = B // QCH


HALF = B // 2


def _emb_body(tab_hbm, idx_hbm, out_hbm, vvec, idxv, outv, vsem):
  wid = lax.axis_index("s") * NC + lax.axis_index("c")

  for k in range(VEC_PW):
    vid = wid * VEC_PW + k
    f = vid // D
    d = lax.rem(vid, D)
    pltpu.async_copy(tab_hbm.at[f, d], vvec, vsem)
    pltpu.sync_copy(idx_hbm.at[f], idxv)
    pltpu.make_async_copy(tab_hbm.at[f, d], vvec, vsem).wait()

    for half in range(2):
      def step(i, _, half=half):
        base = i * 256
        for j in range(16):
          idx16 = idxv[pl.ds(half * HALF + base + j * 16, 16)]
          outv[pl.ds(base + j * 16, 16)] = plsc.load_gather(vvec, [idx16])
        return 0

      lax.fori_loop(0, HALF // 256, step, 0)
      pltpu.sync_copy(outv, out_hbm.at[f, d, pl.ds(half * HALF, HALF)])


@jax.jit
def kernel(indices, tables):
  tabT = jnp.transpose(tables, (0, 2, 1))
  idxT = jnp.transpose(indices, (1, 0))
  mesh = plsc.VectorSubcoreMesh(
      core_axis_name="c", subcore_axis_name="s", num_cores=NC, num_subcores=NS)
  run = functools.partial(
      pl.kernel,
      out_type=jax.ShapeDtypeStruct((F, D, B), jnp.float32),
      mesh=mesh,
      scratch_types=[
          pltpu.VMEM((V,), jnp.float32),
          pltpu.VMEM((B,), jnp.int32),
          pltpu.VMEM((HALF,), jnp.float32),
          pltpu.SemaphoreType.DMA,
      ],
      compiler_params=pltpu.CompilerParams(needs_layout_passes=False),
  )(_emb_body)
  outT = run(tabT, idxT)
  return jnp.transpose(outT, (2, 0, 1))


# speedup vs baseline: 5.3734x; 1.0934x over previous
import functools

import jax
import jax.numpy as jnp
from jax import lax
from jax.experimental import pallas as pl
from jax.experimental.pallas import tpu as pltpu
from jax.experimental.pallas import tpu_sc as plsc

B = 16384
F = 26
V = 100001
D = 32

NC = 2
NS = 16
NW = NC * NS
VEC_PW = F * D // NW

QCH = 4096
NQ = B // QCH


HALF = B // 2


def _emb_body(tab_hbm, idx_hbm, out_hbm, vvec, idxv, outv, vsem):
  wid = lax.axis_index("s") * NC + lax.axis_index("c")

  for k in range(VEC_PW):
    vid = wid * VEC_PW + k
    f = vid // D
    d = lax.rem(vid, D)
    pltpu.async_copy(tab_hbm.at[f, d], vvec, vsem)
    if k == 0:
      pltpu.sync_copy(idx_hbm.at[f], idxv)
    else:
      @pl.when(f != (vid - 1) // D)
      def _():
        pltpu.sync_copy(idx_hbm.at[f], idxv)
    pltpu.make_async_copy(tab_hbm.at[f, d], vvec, vsem).wait()

    for half in range(2):
      def step(i, _, half=half):
        base = i * 256
        for j in range(16):
          idx16 = idxv[pl.ds(half * HALF + base + j * 16, 16)]
          outv[pl.ds(base + j * 16, 16)] = plsc.load_gather(vvec, [idx16])
        return 0

      lax.fori_loop(0, HALF // 256, step, 0)
      pltpu.sync_copy(outv, out_hbm.at[f, d, pl.ds(half * HALF, HALF)])


@jax.jit
def kernel(indices, tables):
  tabT = jnp.transpose(tables, (0, 2, 1))
  idxT = jnp.transpose(indices, (1, 0))
  mesh = plsc.VectorSubcoreMesh(
      core_axis_name="c", subcore_axis_name="s", num_cores=NC, num_subcores=NS)
  run = functools.partial(
      pl.kernel,
      out_type=jax.ShapeDtypeStruct((F, D, B), jnp.float32),
      mesh=mesh,
      scratch_types=[
          pltpu.VMEM((V,), jnp.float32),
          pltpu.VMEM((B,), jnp.int32),
          pltpu.VMEM((HALF,), jnp.float32),
          pltpu.SemaphoreType.DMA,
      ],
      compiler_params=pltpu.CompilerParams(needs_layout_passes=False),
  )(_emb_body)
  outT = run(tabT, idxT)
  return jnp.transpose(outT, (2, 0, 1))
